# Initial kernel scaffold; baseline (speedup 1.0000x reference)
import jax
import jax.numpy as jnp
from jax.experimental import pallas as pl


def kernel(x, W1, b1, W2, b2, codebook, W3, b3, W4, b4):
    raise NotImplementedError("write your pallas kernel here")


# speedup vs baseline: 15.7270x; 15.7270x over previous
import functools

import jax
import jax.numpy as jnp
from jax import lax
from jax.experimental import pallas as pl
from jax.experimental.pallas import tpu as pltpu

B = 1024
IN = 768
H = 512
BLK = 256


def _vqvae_body(x_ref, w1_ref, b1_ref, w2_ref, b2_ref, cb_ref, w3_ref, b3_ref,
                w4_ref, b4_ref, xr_ref, ze_ref, zq_ref):
    x = x_ref[...]
    h = jnp.maximum(
        jnp.dot(x, w1_ref[...], preferred_element_type=jnp.float32) + b1_ref[...], 0.0)
    z_e = jnp.maximum(
        jnp.dot(h, w2_ref[...], preferred_element_type=jnp.float32) + b2_ref[...], 0.0)

    cb = cb_ref[...]
    rs = jnp.sum(cb, axis=1)[None, :]
    q = jnp.sum(cb * cb, axis=1)[None, :]
    scores = jnp.float32(H) * z_e * z_e - 2.0 * z_e * rs + q

    mn = jnp.min(scores, axis=1, keepdims=True)
    iota = lax.broadcasted_iota(jnp.int32, scores.shape, 1)
    idx = jnp.min(jnp.where(scores == mn, iota, H), axis=1, keepdims=True)
    onehot = (iota == idx).astype(jnp.float32)
    z_q = jnp.dot(onehot, cb, preferred_element_type=jnp.float32)

    d = jnp.maximum(
        jnp.dot(z_q, w3_ref[...], preferred_element_type=jnp.float32) + b3_ref[...], 0.0)
    logits = jnp.dot(d, w4_ref[...], preferred_element_type=jnp.float32) + b4_ref[...]
    xr_ref[...] = jax.nn.sigmoid(logits)
    ze_ref[...] = z_e
    zq_ref[...] = z_q


@jax.jit
def kernel(x, W1, b1, W2, b2, codebook, W3, b3, W4, b4):
    grid = (B // BLK,)
    full = lambda shape: pl.BlockSpec(shape, lambda i: (0, 0))
    row_blk = lambda cols: pl.BlockSpec((BLK, cols), lambda i: (i, 0))
    out = pl.pallas_call(
        _vqvae_body,
        grid=grid,
        in_specs=[
            row_blk(IN),
            full((IN, H)), full((1, H)),
            full((H, H)), full((1, H)),
            full((H, H)),
            full((H, H)), full((1, H)),
            full((H, IN)), full((1, IN)),
        ],
        out_specs=[row_blk(IN), row_blk(H), row_blk(H)],
        out_shape=[
            jax.ShapeDtypeStruct((B, IN), jnp.float32),
            jax.ShapeDtypeStruct((B, H), jnp.float32),
            jax.ShapeDtypeStruct((B, H), jnp.float32),
        ],
        compiler_params=pltpu.CompilerParams(
            dimension_semantics=("arbitrary",),
        ),
    )(x, W1, b1.reshape(1, H), W2, b2.reshape(1, H), codebook,
      W3, b3.reshape(1, H), W4, b4.reshape(1, IN))
    return (out[0], out[1], out[2])
